# baseline (speedup 1.0000x reference)
import functools

import jax
import jax.numpy as jnp
from jax import lax
from jax.experimental import pallas as pl
from jax.experimental.pallas import tpu as pltpu
from jax.experimental.pallas import tpu_sc as plsc

VOCAB = 100000
EMBED = 32
BATCH = 1024
CTX = 4


def _make_sc_pool():
    info = plsc.get_sparse_core_info()
    NC, NS, L = info.num_cores, info.num_subcores, info.num_lanes
    NW = NC * NS
    assert BATCH % NW == 0
    b_per_w = BATCH // NW
    idx_per_w = b_per_w * CTX
    mesh = plsc.VectorSubcoreMesh(core_axis_name="c", subcore_axis_name="s")

    @functools.partial(
        pl.kernel,
        mesh=mesh,
        compiler_params=pltpu.CompilerParams(use_tc_tiling_on_sc=False),
        out_type=jax.ShapeDtypeStruct((BATCH, EMBED), jnp.float32),
        scratch_types=[
            pltpu.VMEM((idx_per_w,), jnp.int32),
            pltpu.VMEM((idx_per_w, EMBED), jnp.float32),
            pltpu.VMEM((b_per_w, EMBED), jnp.float32),
            pltpu.SemaphoreType.DMA,
        ],
    )
    def sc_pool(table_hbm, idx_hbm, out_hbm, idx_v, rows_v, pooled_v, sem):
        wid = lax.axis_index("s") * NC + lax.axis_index("c")
        pltpu.sync_copy(idx_hbm.at[pl.ds(wid * idx_per_w, idx_per_w)], idx_v)
        pltpu.async_copy(table_hbm.at[idx_v], rows_v, sem).wait()
        for b in range(b_per_w):
            for c in range(EMBED // L):
                col = pl.ds(c * L, L)
                acc = rows_v[CTX * b, col]
                for k in range(1, CTX):
                    acc = acc + rows_v[CTX * b + k, col]
                pooled_v[b, col] = acc * (1.0 / CTX)
        pltpu.sync_copy(pooled_v, out_hbm.at[pl.ds(wid * b_per_w, b_per_w)])

    return sc_pool


_sc_pool = _make_sc_pool()


VBLK = 512
NBLK = pl.cdiv(VOCAB, VBLK)
TAIL = VOCAB - (NBLK - 1) * VBLK
NBUF = 3
SPLIT = 4
SUB = VBLK // SPLIT
TAIL_SUBS = [min(SUB, max(0, TAIL - i * SUB)) for i in range(SPLIT)]


def _issue(o_buf, o_hbm, sems, slot, j, start, is_last=False):
    for i in range(SPLIT):
        rows = TAIL_SUBS[i] if is_last else SUB
        if rows == 0:
            continue
        cp = pltpu.make_async_copy(
            o_buf.at[slot, pl.ds(i * SUB, rows)],
            o_hbm.at[pl.ds(j * VBLK + i * SUB, rows), :],
            sems.at[slot, i],
        )
        if start:
            cp.start()
        else:
            cp.wait()


def _mm_kernel(wt_ref, p_ref, b_ref, o_hbm, o_buf, sems):
    j = pl.program_id(0)
    slot = lax.rem(j, NBUF)

    @pl.when(j >= NBUF)
    def _wait_prev():
        _issue(o_buf, o_hbm, sems, slot, j - NBUF, start=False)

    o_buf[slot] = lax.dot_general(
        wt_ref[...], p_ref[...],
        dimension_numbers=(((0,), (1,)), ((), ())),
        preferred_element_type=jnp.float32,
    ) + b_ref[...]

    @pl.when(j < NBLK - 1)
    def _start_full():
        _issue(o_buf, o_hbm, sems, slot, j, start=True)

    @pl.when(j == NBLK - 1)
    def _last():
        _issue(o_buf, o_hbm, sems, slot, NBLK - 1, start=True, is_last=True)
        for k in range(NBUF):
            g = NBLK - NBUF + k
            _issue(o_buf, o_hbm, sems, g % NBUF, g, start=False,
                   is_last=(g == NBLK - 1))


def _project_t(W_out_w_t, pooled, bias_col):
    return pl.pallas_call(
        _mm_kernel,
        grid=(NBLK,),
        in_specs=[
            pl.BlockSpec((EMBED, VBLK), lambda j: (0, j)),
            pl.BlockSpec((BATCH, EMBED), lambda j: (0, 0)),
            pl.BlockSpec((VBLK, 1), lambda j: (j, 0)),
        ],
        out_specs=pl.BlockSpec(memory_space=pl.ANY),
        out_shape=jax.ShapeDtypeStruct((VOCAB, BATCH), jnp.float32),
        scratch_shapes=[
            pltpu.VMEM((NBUF, VBLK, BATCH), jnp.float32),
            pltpu.SemaphoreType.DMA((NBUF, SPLIT)),
        ],
        compiler_params=pltpu.CompilerParams(
            dimension_semantics=("arbitrary",),
        ),
    )(W_out_w_t, pooled, bias_col)


@jax.jit
def kernel(context_words, W_in, W_out_w, W_out_b):
    idx = context_words.reshape(-1).astype(jnp.int32)
    pooled = _sc_pool(W_in, idx)
    out_t = _project_t(W_out_w.T, pooled, W_out_b.reshape(VOCAB, 1))
    return out_t.T


# speedup vs baseline: 1.2419x; 1.2419x over previous
import functools

import jax
import jax.numpy as jnp
from jax import lax
from jax.experimental import pallas as pl
from jax.experimental.pallas import tpu as pltpu
from jax.experimental.pallas import tpu_sc as plsc

VOCAB = 100000
EMBED = 32
BATCH = 1024
CTX = 4


def _make_sc_pool():
    info = plsc.get_sparse_core_info()
    NC, NS, L = info.num_cores, info.num_subcores, info.num_lanes
    NW = NC * NS
    assert BATCH % NW == 0
    b_per_w = BATCH // NW
    idx_per_w = b_per_w * CTX
    mesh = plsc.VectorSubcoreMesh(core_axis_name="c", subcore_axis_name="s")

    @functools.partial(
        pl.kernel,
        mesh=mesh,
        compiler_params=pltpu.CompilerParams(use_tc_tiling_on_sc=False),
        out_type=jax.ShapeDtypeStruct((BATCH, EMBED), jnp.float32),
        scratch_types=[
            pltpu.VMEM((idx_per_w,), jnp.int32),
            pltpu.VMEM((idx_per_w, EMBED), jnp.float32),
            pltpu.VMEM((b_per_w, EMBED), jnp.float32),
            pltpu.SemaphoreType.DMA,
        ],
    )
    def sc_pool(table_hbm, idx_hbm, out_hbm, idx_v, rows_v, pooled_v, sem):
        wid = lax.axis_index("s") * NC + lax.axis_index("c")
        pltpu.sync_copy(idx_hbm.at[pl.ds(wid * idx_per_w, idx_per_w)], idx_v)
        pltpu.async_copy(table_hbm.at[idx_v], rows_v, sem).wait()
        for b in range(b_per_w):
            for c in range(EMBED // L):
                col = pl.ds(c * L, L)
                acc = rows_v[CTX * b, col]
                for k in range(1, CTX):
                    acc = acc + rows_v[CTX * b + k, col]
                pooled_v[b, col] = acc * (1.0 / CTX)
        pltpu.sync_copy(pooled_v, out_hbm.at[pl.ds(wid * b_per_w, b_per_w)])

    return sc_pool


_sc_pool = _make_sc_pool()


VBLK = 3072
NBLK = pl.cdiv(VOCAB, VBLK)
TAIL = VOCAB - (NBLK - 1) * VBLK
NBUF = 3
SPLIT = 4
SUB = VBLK // SPLIT
TAIL_SUBS = [min(SUB, max(0, TAIL - i * SUB)) for i in range(SPLIT)]


def _issue(o_buf, o_hbm, sems, slot, j, start, is_last=False):
    for i in range(SPLIT):
        rows = TAIL_SUBS[i] if is_last else SUB
        if rows == 0:
            continue
        cp = pltpu.make_async_copy(
            o_buf.at[slot, pl.ds(i * SUB, rows)],
            o_hbm.at[pl.ds(j * VBLK + i * SUB, rows), :],
            sems.at[slot, i],
        )
        if start:
            cp.start()
        else:
            cp.wait()


def _mm_kernel(wt_ref, p_ref, b_ref, o_hbm, o_buf, sems):
    j = pl.program_id(0)
    slot = lax.rem(j, NBUF)

    @pl.when(j >= NBUF)
    def _wait_prev():
        _issue(o_buf, o_hbm, sems, slot, j - NBUF, start=False)

    o_buf[slot] = lax.dot_general(
        wt_ref[...], p_ref[...],
        dimension_numbers=(((0,), (1,)), ((), ())),
        preferred_element_type=jnp.float32,
    ) + b_ref[...]

    @pl.when(j < NBLK - 1)
    def _start_full():
        _issue(o_buf, o_hbm, sems, slot, j, start=True)

    @pl.when(j == NBLK - 1)
    def _last():
        _issue(o_buf, o_hbm, sems, slot, NBLK - 1, start=True, is_last=True)
        for k in range(NBUF):
            g = NBLK - NBUF + k
            _issue(o_buf, o_hbm, sems, g % NBUF, g, start=False,
                   is_last=(g == NBLK - 1))


def _project_t(W_out_w_t, pooled, bias_col):
    return pl.pallas_call(
        _mm_kernel,
        grid=(NBLK,),
        in_specs=[
            pl.BlockSpec((EMBED, VBLK), lambda j: (0, j)),
            pl.BlockSpec((BATCH, EMBED), lambda j: (0, 0)),
            pl.BlockSpec((VBLK, 1), lambda j: (j, 0)),
        ],
        out_specs=pl.BlockSpec(memory_space=pl.ANY),
        out_shape=jax.ShapeDtypeStruct((VOCAB, BATCH), jnp.float32),
        scratch_shapes=[
            pltpu.VMEM((NBUF, VBLK, BATCH), jnp.float32),
            pltpu.SemaphoreType.DMA((NBUF, SPLIT)),
        ],
        compiler_params=pltpu.CompilerParams(
            dimension_semantics=("arbitrary",),
        ),
    )(W_out_w_t, pooled, bias_col)


@jax.jit
def kernel(context_words, W_in, W_out_w, W_out_b):
    idx = context_words.reshape(-1).astype(jnp.int32)
    pooled = _sc_pool(W_in, idx)
    out_t = _project_t(W_out_w.T, pooled, W_out_b.reshape(VOCAB, 1))
    return out_t.T


# speedup vs baseline: 1.2509x; 1.0072x over previous
import functools

import jax
import jax.numpy as jnp
from jax import lax
from jax.experimental import pallas as pl
from jax.experimental.pallas import tpu as pltpu
from jax.experimental.pallas import tpu_sc as plsc

VOCAB = 100000
EMBED = 32
BATCH = 1024
CTX = 4


def _make_sc_pool():
    info = plsc.get_sparse_core_info()
    NC, NS, L = info.num_cores, info.num_subcores, info.num_lanes
    NW = NC * NS
    assert BATCH % NW == 0
    b_per_w = BATCH // NW
    idx_per_w = b_per_w * CTX
    mesh = plsc.VectorSubcoreMesh(core_axis_name="c", subcore_axis_name="s")

    @functools.partial(
        pl.kernel,
        mesh=mesh,
        compiler_params=pltpu.CompilerParams(use_tc_tiling_on_sc=False),
        out_type=jax.ShapeDtypeStruct((BATCH, EMBED), jnp.float32),
        scratch_types=[
            pltpu.VMEM((idx_per_w,), jnp.int32),
            pltpu.VMEM((idx_per_w, EMBED), jnp.float32),
            pltpu.VMEM((b_per_w, EMBED), jnp.float32),
            pltpu.SemaphoreType.DMA,
        ],
    )
    def sc_pool(table_hbm, idx_hbm, out_hbm, idx_v, rows_v, pooled_v, sem):
        wid = lax.axis_index("s") * NC + lax.axis_index("c")
        pltpu.sync_copy(idx_hbm.at[pl.ds(wid * idx_per_w, idx_per_w)], idx_v)
        pltpu.async_copy(table_hbm.at[idx_v], rows_v, sem).wait()
        for b in range(b_per_w):
            for c in range(EMBED // L):
                col = pl.ds(c * L, L)
                acc = rows_v[CTX * b, col]
                for k in range(1, CTX):
                    acc = acc + rows_v[CTX * b + k, col]
                pooled_v[b, col] = acc * (1.0 / CTX)
        pltpu.sync_copy(pooled_v, out_hbm.at[pl.ds(wid * b_per_w, b_per_w)])

    return sc_pool


_sc_pool = _make_sc_pool()


VBLK = 4096
NBLK = pl.cdiv(VOCAB, VBLK)
TAIL = VOCAB - (NBLK - 1) * VBLK
NBUF = 2
SPLIT = 4
SUB = VBLK // SPLIT
TAIL_SUBS = [min(SUB, max(0, TAIL - i * SUB)) for i in range(SPLIT)]


def _issue(o_buf, o_hbm, sems, slot, j, start, is_last=False):
    for i in range(SPLIT):
        rows = TAIL_SUBS[i] if is_last else SUB
        if rows == 0:
            continue
        cp = pltpu.make_async_copy(
            o_buf.at[slot, pl.ds(i * SUB, rows)],
            o_hbm.at[pl.ds(j * VBLK + i * SUB, rows), :],
            sems.at[slot, i],
        )
        if start:
            cp.start()
        else:
            cp.wait()


def _mm_kernel(wt_ref, p_ref, b_ref, o_hbm, o_buf, sems):
    j = pl.program_id(0)
    slot = lax.rem(j, NBUF)

    @pl.when(j >= NBUF)
    def _wait_prev():
        _issue(o_buf, o_hbm, sems, slot, j - NBUF, start=False)

    o_buf[slot] = lax.dot_general(
        wt_ref[...], p_ref[...],
        dimension_numbers=(((0,), (1,)), ((), ())),
        preferred_element_type=jnp.float32,
    ) + b_ref[...]

    @pl.when(j < NBLK - 1)
    def _start_full():
        _issue(o_buf, o_hbm, sems, slot, j, start=True)

    @pl.when(j == NBLK - 1)
    def _last():
        _issue(o_buf, o_hbm, sems, slot, NBLK - 1, start=True, is_last=True)
        for k in range(NBUF):
            g = NBLK - NBUF + k
            _issue(o_buf, o_hbm, sems, g % NBUF, g, start=False,
                   is_last=(g == NBLK - 1))


def _project_t(W_out_w_t, pooled, bias_col):
    return pl.pallas_call(
        _mm_kernel,
        grid=(NBLK,),
        in_specs=[
            pl.BlockSpec((EMBED, VBLK), lambda j: (0, j)),
            pl.BlockSpec((BATCH, EMBED), lambda j: (0, 0)),
            pl.BlockSpec((VBLK, 1), lambda j: (j, 0)),
        ],
        out_specs=pl.BlockSpec(memory_space=pl.ANY),
        out_shape=jax.ShapeDtypeStruct((VOCAB, BATCH), jnp.float32),
        scratch_shapes=[
            pltpu.VMEM((NBUF, VBLK, BATCH), jnp.float32),
            pltpu.SemaphoreType.DMA((NBUF, SPLIT)),
        ],
        compiler_params=pltpu.CompilerParams(
            dimension_semantics=("arbitrary",),
        ),
    )(W_out_w_t, pooled, bias_col)


@jax.jit
def kernel(context_words, W_in, W_out_w, W_out_b):
    idx = context_words.reshape(-1).astype(jnp.int32)
    pooled = _sc_pool(W_in, idx)
    out_t = _project_t(W_out_w.T, pooled, W_out_b.reshape(VOCAB, 1))
    return out_t.T
